# Initial kernel scaffold; baseline (speedup 1.0000x reference)
import jax
import jax.numpy as jnp
from jax.experimental import pallas as pl


def kernel(features, weights, adjacency, w_selected, w_nbweights_ew, w_nbweights, w_nbpriors, w_q_allembed, w_q_action, w_q_reduc):
    raise NotImplementedError("write your pallas kernel here")


# speedup vs baseline: 1.9876x; 1.9876x over previous
from functools import partial

import jax
import jax.numpy as jnp
from jax import lax
from jax.experimental import pallas as pl
from jax.experimental.pallas import tpu as pltpu


def _ge_kernel(f_ref, w_ref, a_ref, p_ref, q_ref, emb_ref, *, iters, G, E):
    wnbp_t = p_ref[0:E, :].astype(jnp.bfloat16)
    vecs = p_ref[E:E + 8, :]
    wpos = vecs[0:1, :]
    wneg = vecs[1:2, :]
    v_all = vecs[2:3, :]
    v_act = vecs[3:4, :]
    wsel = vecs[4:5, :]

    w = w_ref[0]
    pos = jnp.sum(jnp.maximum(w, 0.0), axis=1, keepdims=True)
    neg = jnp.sum(jnp.maximum(-w, 0.0), axis=1, keepdims=True)
    f_col = f_ref[...]
    base = f_col * wsel + pos * wpos + neg * wneg

    adj = a_ref[0].astype(jnp.bfloat16)

    if iters <= 0:
        emb = jnp.zeros_like(base)
    else:
        def body(_, emb):
            vp = jnp.dot(adj, emb.astype(jnp.bfloat16),
                         preferred_element_type=jnp.float32)
            vp = jnp.dot(vp.astype(jnp.bfloat16), wnbp_t,
                         preferred_element_type=jnp.float32)
            return jnp.maximum(base + vp, 0.0)

        emb = lax.fori_loop(0, iters - 1, body, jnp.maximum(base, 0.0))

    emb_ref[...] = emb

    sum_g = jnp.sum(emb, axis=0, keepdims=True)
    t_act = jnp.sum(emb * v_act, axis=1, keepdims=True)
    t_all = jnp.sum(sum_g * v_all, axis=1, keepdims=True)
    q_ref[...] = t_act + t_all


@partial(jax.jit, static_argnames=("iters",))
def _graph_embedder(features, weights, adjacency, params, iters=5):
    wsel, wew, wnbw, wnbp, wqall, wqact, wreduc = params
    B, G = features.shape
    E = wsel.shape[0]

    f = features.astype(jnp.float32).reshape(B * G, 1)
    w = weights.astype(jnp.float32)
    a = adjacency.astype(jnp.float32)

    wsel_row = wsel[:, 0].astype(jnp.float32)
    wpos = jnp.sum(wnbw * jnp.maximum(wew[:, 0], 0.0)[None, :], axis=1)
    wneg = jnp.sum(wnbw * jnp.maximum(-wew[:, 0], 0.0)[None, :], axis=1)
    v_all = jnp.sum(wqall * wreduc[0, :E][:, None], axis=0)
    v_act = jnp.sum(wqact * wreduc[0, E:][:, None], axis=0)
    packed = jnp.concatenate(
        [wnbp.T.astype(jnp.float32),
         jnp.stack([wpos, wneg, v_all, v_act, wsel_row], axis=0),
         jnp.zeros((3, E), jnp.float32)], axis=0).astype(jnp.float32)

    kern = partial(_ge_kernel, iters=iters, G=G, E=E)

    q_flat, emb_flat = pl.pallas_call(
        kern,
        out_shape=(jax.ShapeDtypeStruct((B * G, 1), jnp.float32),
                   jax.ShapeDtypeStruct((B * G, E), jnp.float32)),
        grid_spec=pltpu.PrefetchScalarGridSpec(
            num_scalar_prefetch=0,
            grid=(B,),
            in_specs=[
                pl.BlockSpec((G, 1), lambda b: (b, 0)),
                pl.BlockSpec((1, G, G), lambda b: (b, 0, 0)),
                pl.BlockSpec((1, G, G), lambda b: (b, 0, 0)),
                pl.BlockSpec((E + 8, E), lambda b: (0, 0)),
            ],
            out_specs=[
                pl.BlockSpec((G, 1), lambda b: (b, 0)),
                pl.BlockSpec((G, E), lambda b: (b, 0)),
            ],
        ),
        compiler_params=pltpu.CompilerParams(
            dimension_semantics=("parallel",),
            vmem_limit_bytes=64 * 1024 * 1024),
    )(f, w, a, packed)

    q = q_flat.reshape(B, G)
    emb = emb_flat.reshape(B, G, E)
    return q, emb


def kernel(features, weights, adjacency, w_selected, w_nbweights_ew,
           w_nbweights, w_nbpriors, w_q_allembed, w_q_action, w_q_reduc):
    params = (w_selected, w_nbweights_ew, w_nbweights, w_nbpriors,
              w_q_allembed, w_q_action, w_q_reduc)
    return _graph_embedder(features, weights, adjacency, params, iters=5)
